# baseline (speedup 1.0000x reference)
import functools

import jax
import jax.numpy as jnp
from jax import lax
from jax.experimental import pallas as pl
from jax.experimental.pallas import tpu as pltpu
from jax.experimental.pallas import tpu_sc as plsc

B = 16384
M = 32
L = 16
NC = 2
NS = 16
NW = NC * NS
BPW = B // NW
CH = 128
NCH = BPW // CH
IDX_ROWS = B // CH


def _sc_mf(user2d, item2d, ue_t, ie_t, ubias, ibias, offset):
  mesh = plsc.VectorSubcoreMesh(core_axis_name="c", subcore_axis_name="s")
  f32 = jnp.float32

  @functools.partial(
      pl.kernel,
      out_type=jax.ShapeDtypeStruct((B,), f32),
      mesh=mesh,
      compiler_params=pltpu.CompilerParams(use_tc_tiling_on_sc=False),
      scratch_types=[
          pltpu.VMEM((NCH, CH), jnp.int32),
          pltpu.VMEM((NCH, CH), jnp.int32),
          pltpu.VMEM((M * BPW,), f32),
          pltpu.VMEM((M * BPW,), f32),
          pltpu.VMEM((BPW,), f32),
          pltpu.VMEM((BPW,), f32),
          pltpu.VMEM((BPW,), f32),
          pltpu.VMEM((L,), f32),
          pltpu.SemaphoreType.DMA,
      ],
  )
  def k(user_hbm, item_hbm, ue_hbm, ie_hbm, ubias_hbm, ibias_hbm, off_hbm,
        out_hbm, uidx_v, iidx_v, uflat, vflat, ub_v, ib_v, outv, off_v, sem):
    cid = lax.axis_index("c")
    sid = lax.axis_index("s")
    wid = sid * NC + cid
    base = wid * BPW
    rowbase = wid * NCH
    pltpu.sync_copy(user_hbm.at[pl.ds(rowbase, NCH)], uidx_v)
    pltpu.sync_copy(item_hbm.at[pl.ds(rowbase, NCH)], iidx_v)

    for j in range(NCH):
      for c in range(M):
        pltpu.async_copy(
            ue_hbm.at[c].at[uidx_v.at[j]],
            uflat.at[pl.ds(c * BPW + j * CH, CH)], sem)
        pltpu.async_copy(
            ie_hbm.at[c].at[iidx_v.at[j]],
            vflat.at[pl.ds(c * BPW + j * CH, CH)], sem)
      pltpu.async_copy(
          ubias_hbm.at[uidx_v.at[j]], ub_v.at[pl.ds(j * CH, CH)], sem)
      pltpu.async_copy(
          ibias_hbm.at[iidx_v.at[j]], ib_v.at[pl.ds(j * CH, CH)], sem)

    pltpu.sync_copy(off_hbm, off_v)
    pltpu.make_async_copy(ubias_hbm.at[pl.ds(0, M * BPW)], uflat, sem).wait()
    pltpu.make_async_copy(ubias_hbm.at[pl.ds(0, M * BPW)], vflat, sem).wait()
    pltpu.make_async_copy(ubias_hbm.at[pl.ds(0, BPW)], ub_v, sem).wait()
    pltpu.make_async_copy(ubias_hbm.at[pl.ds(0, BPW)], ib_v, sem).wait()

    @pl.loop(0, BPW, step=L)
    def _(s):
      acc = uflat[pl.ds(s, L)] * vflat[pl.ds(s, L)]
      for c in range(1, M):
        acc = acc + (uflat[pl.ds(c * BPW + s, L)]
                     * vflat[pl.ds(c * BPW + s, L)])
      outv[pl.ds(s, L)] = (acc + ub_v[pl.ds(s, L)] + ib_v[pl.ds(s, L)]
                           + off_v[pl.ds(0, L)])

    pltpu.sync_copy(outv, out_hbm.at[pl.ds(base, BPW)])

  return k(user2d, item2d, ue_t, ie_t, ubias, ibias, offset)


@jax.jit
def kernel(user, item, user_emb, item_emb, user_bias, item_bias, offset):
  user = user.astype(jnp.int32)
  item = item.astype(jnp.int32)
  offset_b = jnp.broadcast_to(offset, (L,))
  return _sc_mf(
      user.reshape(IDX_ROWS, CH), item.reshape(IDX_ROWS, CH),
      user_emb.T, item_emb.T, user_bias, item_bias, offset_b)


# speedup vs baseline: 5.5067x; 5.5067x over previous
import functools

import jax
import jax.numpy as jnp
from jax import lax
from jax.experimental import pallas as pl
from jax.experimental.pallas import tpu as pltpu
from jax.experimental.pallas import tpu_sc as plsc

B = 16384
M = 32
NC = 2
NS = 16
NW = NC * NS
BPW = B // NW
CH = 128
NCH = BPW // CH
IDX_ROWS = B // CH


def _sc_gather(user2d, item2d, user_emb, item_emb, user_bias, item_bias):
  mesh = plsc.VectorSubcoreMesh(core_axis_name="c", subcore_axis_name="s")
  f32 = jnp.float32
  out_type = (
      jax.ShapeDtypeStruct((B, M), f32),
      jax.ShapeDtypeStruct((B, M), f32),
      jax.ShapeDtypeStruct((IDX_ROWS, CH), f32),
      jax.ShapeDtypeStruct((IDX_ROWS, CH), f32),
  )

  @functools.partial(
      pl.kernel,
      out_type=out_type,
      mesh=mesh,
      compiler_params=pltpu.CompilerParams(use_tc_tiling_on_sc=False),
      scratch_types=[
          pltpu.VMEM((NCH, CH), jnp.int32),
          pltpu.VMEM((NCH, CH), jnp.int32),
          pltpu.VMEM((BPW, M), f32),
          pltpu.VMEM((BPW, M), f32),
          pltpu.VMEM((NCH, CH), f32),
          pltpu.VMEM((NCH, CH), f32),
          pltpu.SemaphoreType.DMA,
      ],
  )
  def k(user_hbm, item_hbm, uemb_hbm, iemb_hbm, ubias_hbm, ibias_hbm,
        u_out, v_out, ub_out, ib_out,
        uidx_v, iidx_v, u_v, v_v, ub_v, ib_v, sem):
    wid = lax.axis_index("s") * NC + lax.axis_index("c")
    rowbase = wid * NCH
    pltpu.sync_copy(user_hbm.at[pl.ds(rowbase, NCH)], uidx_v)
    pltpu.sync_copy(item_hbm.at[pl.ds(rowbase, NCH)], iidx_v)
    copies = []
    for j in range(NCH):
      dst = pl.ds(j * CH, CH)
      copies.append(pltpu.async_copy(
          uemb_hbm.at[uidx_v.at[j]], u_v.at[dst], sem))
      copies.append(pltpu.async_copy(
          iemb_hbm.at[iidx_v.at[j]], v_v.at[dst], sem))
      copies.append(pltpu.async_copy(
          ubias_hbm.at[uidx_v.at[j]], ub_v.at[j], sem))
      copies.append(pltpu.async_copy(
          ibias_hbm.at[iidx_v.at[j]], ib_v.at[j], sem))
    for c in copies:
      c.wait()
    base = wid * BPW
    pltpu.sync_copy(u_v, u_out.at[pl.ds(base, BPW)])
    pltpu.sync_copy(v_v, v_out.at[pl.ds(base, BPW)])
    pltpu.sync_copy(ub_v, ub_out.at[pl.ds(rowbase, NCH)])
    pltpu.sync_copy(ib_v, ib_out.at[pl.ds(rowbase, NCH)])

  return k(user2d, item2d, user_emb, item_emb, user_bias, item_bias)


TC_BLK = 2048


def _tc_dot(u, v, ub, ib, offset):
  def body(u_ref, v_ref, ub_ref, ib_ref, off_ref, o_ref):
    prod = jnp.sum(u_ref[...] * v_ref[...], axis=1)
    o_ref[...] = prod + ub_ref[...] + ib_ref[...] + off_ref[...]

  grid = (B // TC_BLK,)
  return pl.pallas_call(
      body,
      grid=grid,
      in_specs=[
          pl.BlockSpec((TC_BLK, M), lambda i: (i, 0)),
          pl.BlockSpec((TC_BLK, M), lambda i: (i, 0)),
          pl.BlockSpec((TC_BLK,), lambda i: (i,)),
          pl.BlockSpec((TC_BLK,), lambda i: (i,)),
          pl.BlockSpec((1,), lambda i: (0,)),
      ],
      out_specs=pl.BlockSpec((TC_BLK,), lambda i: (i,)),
      out_shape=jax.ShapeDtypeStruct((B,), jnp.float32),
  )(u, v, ub, ib, offset)


@jax.jit
def kernel(user, item, user_emb, item_emb, user_bias, item_bias, offset):
  user = user.astype(jnp.int32)
  item = item.astype(jnp.int32)
  user2d = user.reshape(IDX_ROWS, CH)
  item2d = item.reshape(IDX_ROWS, CH)
  u_g, v_g, ub2, ib2 = _sc_gather(
      user2d, item2d, user_emb, item_emb, user_bias, item_bias)
  return _tc_dot(u_g, v_g, ub2.reshape(B), ib2.reshape(B), offset)


# speedup vs baseline: 7.8647x; 1.4282x over previous
import dataclasses
import functools

import jax
import jax.numpy as jnp
from jax import lax
from jax.experimental import pallas as pl
from jax.experimental.pallas import tpu as pltpu
from jax.experimental.pallas import tpu_sc as plsc

B = 16384
M = 32
L = 16
NC = 2
NS = 16
NW = NC * NS
BPW = B // NW
G = 16
NG = BPW // G
CH = 128
NCH = BPW // CH
IDX_ROWS = B // CH


def _sc_dot(user_flat, item_flat, ue, ie):
  mesh = plsc.VectorSubcoreMesh(core_axis_name="c", subcore_axis_name="s")
  f32 = jnp.float32
  out_type = (
      jax.ShapeDtypeStruct((B,), f32),
      jax.ShapeDtypeStruct((G, 8, M), f32),
  )

  cp = pltpu.CompilerParams()
  if "needs_layout_passes" in pltpu.CompilerParams.__dataclass_fields__:
    cp = dataclasses.replace(cp, needs_layout_passes=False)

  @functools.partial(
      pl.kernel,
      out_type=out_type,
      mesh=mesh,
      compiler_params=cp,
      scratch_types=[
          pltpu.VMEM((BPW,), jnp.int32),
          pltpu.VMEM((BPW,), jnp.int32),
          pltpu.VMEM((G, 8, M), f32),
          pltpu.VMEM((G, 8, M), f32),
          pltpu.VMEM((G, 8, M), f32),
          pltpu.VMEM((G, 8, M), f32),
          pltpu.VMEM((BPW,), f32),
          pltpu.SemaphoreType.DMA,
          pltpu.SemaphoreType.DMA,
      ],
  )
  def k(user_hbm, item_hbm, ue_hbm, ie_hbm, dot_out, dummy,
        uidx_s, iidx_s, ua, ub, va, vb, dotv, sema, semb):
    cid = lax.axis_index("c")
    sid = lax.axis_index("s")
    wid = sid * NC + cid
    base = wid * BPW
    pltpu.sync_copy(user_hbm.at[pl.ds(base, BPW)], uidx_s)
    pltpu.sync_copy(item_hbm.at[pl.ds(base, BPW)], iidx_s)

    def fire(g, ubuf, vbuf, sem):
      idxu = uidx_s[pl.ds(g * G, G)]
      idxi = iidx_s[pl.ds(g * G, G)]
      for jj in range(G):
        u = idxu[jj]
        i = idxi[jj]
        u8 = pl.multiple_of((u >> 3) * 8, 8)
        i8 = pl.multiple_of((i >> 3) * 8, 8)
        pltpu.async_copy(ue_hbm.at[pl.ds(u8, 8), :], ubuf.at[jj], sem)
        pltpu.async_copy(ie_hbm.at[pl.ds(i8, 8), :], vbuf.at[jj], sem)

    def drain(ubuf, vbuf, sem):
      pltpu.make_async_copy(dummy, ubuf, sem).wait()
      pltpu.make_async_copy(dummy, vbuf, sem).wait()

    def extract(g, ubuf, vbuf):
      lane = lax.broadcasted_iota(jnp.int32, (L,), 0)
      acc0 = jnp.zeros((L,), f32)
      idxu = uidx_s[pl.ds(g * G, G)]
      idxi = iidx_s[pl.ds(g * G, G)]
      for jj in range(G):
        su = idxu[jj] & 7
        si = idxi[jj] & 7
        t = (ubuf[jj, su, pl.ds(0, L)] * vbuf[jj, si, pl.ds(0, L)]
             + ubuf[jj, su, pl.ds(L, L)] * vbuf[jj, si, pl.ds(L, L)])
        r = jnp.sum(t)
        acc0 = jnp.where(lane == jj, r, acc0)
      dotv[pl.ds(g * G, L)] = acc0

    fire(0, ua, va, sema)

    @pl.loop(0, NG, step=2)
    def _(g):
      @pl.when(g + 1 < NG)
      def _():
        fire(g + 1, ub, vb, semb)
      drain(ua, va, sema)
      extract(g, ua, va)

      @pl.when(g + 2 < NG)
      def _():
        fire(g + 2, ua, va, sema)

      @pl.when(g + 1 < NG)
      def _():
        drain(ub, vb, semb)
        extract(g + 1, ub, vb)

    pltpu.sync_copy(dotv, dot_out.at[pl.ds(base, BPW)])

  return k(user_flat, item_flat, ue, ie)[0]


def _sc_finish(user2d, item2d, dot, user_bias, item_bias, offset_b):
  mesh = plsc.VectorSubcoreMesh(core_axis_name="c", subcore_axis_name="s")
  f32 = jnp.float32

  @functools.partial(
      pl.kernel,
      out_type=jax.ShapeDtypeStruct((B,), f32),
      mesh=mesh,
      compiler_params=pltpu.CompilerParams(use_tc_tiling_on_sc=False),
      scratch_types=[
          pltpu.VMEM((NCH, CH), jnp.int32),
          pltpu.VMEM((NCH, CH), jnp.int32),
          pltpu.VMEM((BPW,), f32),
          pltpu.VMEM((BPW,), f32),
          pltpu.VMEM((BPW,), f32),
          pltpu.VMEM((BPW,), f32),
          pltpu.VMEM((L,), f32),
          pltpu.SemaphoreType.DMA,
      ],
  )
  def k(user_hbm, item_hbm, dot_hbm, ubias_hbm, ibias_hbm, off_hbm, out_hbm,
        uidx_v, iidx_v, ub_v, ib_v, dv_v, outv, off_v, sem):
    cid = lax.axis_index("c")
    sid = lax.axis_index("s")
    wid = sid * NC + cid
    base = wid * BPW
    rowbase = wid * NCH
    pltpu.sync_copy(user_hbm.at[pl.ds(rowbase, NCH)], uidx_v)
    pltpu.sync_copy(item_hbm.at[pl.ds(rowbase, NCH)], iidx_v)
    copies = [pltpu.async_copy(dot_hbm.at[pl.ds(base, BPW)], dv_v, sem)]
    for j in range(NCH):
      copies.append(pltpu.async_copy(
          ubias_hbm.at[uidx_v.at[j]], ub_v.at[pl.ds(j * CH, CH)], sem))
      copies.append(pltpu.async_copy(
          ibias_hbm.at[iidx_v.at[j]], ib_v.at[pl.ds(j * CH, CH)], sem))
    pltpu.sync_copy(off_hbm, off_v)
    for c in copies:
      c.wait()

    @pl.loop(0, BPW, step=L)
    def _(s):
      outv[pl.ds(s, L)] = (dv_v[pl.ds(s, L)] + ub_v[pl.ds(s, L)]
                           + ib_v[pl.ds(s, L)] + off_v[pl.ds(0, L)])

    pltpu.sync_copy(outv, out_hbm.at[pl.ds(base, BPW)])

  return k(user2d, item2d, dot, user_bias, item_bias, offset_b)


@jax.jit
def kernel(user, item, user_emb, item_emb, user_bias, item_bias, offset):
  user = user.astype(jnp.int32)
  item = item.astype(jnp.int32)
  dot = _sc_dot(user, item, user_emb, item_emb)
  offset_b = jnp.broadcast_to(offset, (L,))
  return _sc_finish(
      user.reshape(IDX_ROWS, CH), item.reshape(IDX_ROWS, CH),
      dot, user_bias, item_bias, offset_b)


# speedup vs baseline: 20.3937x; 2.5931x over previous
import dataclasses
import functools

import jax
import jax.numpy as jnp
from jax import lax
from jax.experimental import pallas as pl
from jax.experimental.pallas import tpu as pltpu
from jax.experimental.pallas import tpu_sc as plsc

B = 16384
M = 32
L = 16
NC = 2
NS = 16
NW = NC * NS
BPW = B // NW
G = 16
G4 = 4
NG = BPW // G
CH = 128
NCH = BPW // CH
IDX_ROWS = B // CH


def _sc_dot(user_flat, item_flat, ue, ie):
  mesh = plsc.VectorSubcoreMesh(core_axis_name="c", subcore_axis_name="s")
  f32 = jnp.float32
  out_type = (
      jax.ShapeDtypeStruct((B,), f32),
      jax.ShapeDtypeStruct((G4, M, 128), f32),
  )

  cp = pltpu.CompilerParams()
  if "needs_layout_passes" in pltpu.CompilerParams.__dataclass_fields__:
    cp = dataclasses.replace(cp, needs_layout_passes=False)

  @functools.partial(
      pl.kernel,
      out_type=out_type,
      mesh=mesh,
      compiler_params=cp,
      scratch_types=[
          pltpu.VMEM((BPW,), jnp.int32),
          pltpu.VMEM((BPW,), jnp.int32),
          pltpu.VMEM((G4, M, 128), f32),
          pltpu.VMEM((G4, M, 128), f32),
          pltpu.VMEM((G4, M, 128), f32),
          pltpu.VMEM((G4, M, 128), f32),
          pltpu.VMEM((BPW,), f32),
          pltpu.SemaphoreType.DMA,
          pltpu.SemaphoreType.DMA,
      ],
  )
  def k(user_hbm, item_hbm, ue_hbm, ie_hbm, dot_out, dummy,
        uidx_s, iidx_s, ua, ub, va, vb, dotv, sema, semb):
    cid = lax.axis_index("c")
    sid = lax.axis_index("s")
    wid = sid * NC + cid
    base = wid * BPW
    pltpu.sync_copy(user_hbm.at[pl.ds(base, BPW)], uidx_s)
    pltpu.sync_copy(item_hbm.at[pl.ds(base, BPW)], iidx_s)

    def fire(ws, pos, ubuf, vbuf, sem):
      idxu = uidx_s[pl.ds(ws, L)]
      idxi = iidx_s[pl.ds(ws, L)]
      for jj in range(G4):
        u = idxu[pos * G4 + jj]
        i = idxi[pos * G4 + jj]
        u128 = pl.multiple_of((u >> 7) * 128, 128)
        i128 = pl.multiple_of((i >> 7) * 128, 128)
        pltpu.async_copy(ue_hbm.at[:, pl.ds(u128, 128)], ubuf.at[jj], sem)
        pltpu.async_copy(ie_hbm.at[:, pl.ds(i128, 128)], vbuf.at[jj], sem)

    def drain(ubuf, vbuf, sem):
      pltpu.make_async_copy(dummy, ubuf, sem).wait()
      pltpu.make_async_copy(dummy, vbuf, sem).wait()

    rows0 = lax.broadcasted_iota(jnp.int32, (L,), 0)
    rows1 = rows0 + L

    def extract(ws, pos, ubuf, vbuf, acc0, lane):
      idxu = uidx_s[pl.ds(ws, L)]
      idxi = iidx_s[pl.ds(ws, L)]
      for jj in range(G4):
        b = pos * G4 + jj
        ucols = jnp.full((L,), idxu[b] & 127, jnp.int32)
        icols = jnp.full((L,), idxi[b] & 127, jnp.int32)
        ua0 = plsc.load_gather(ubuf.at[jj], [rows0, ucols])
        ua1 = plsc.load_gather(ubuf.at[jj], [rows1, ucols])
        va0 = plsc.load_gather(vbuf.at[jj], [rows0, icols])
        va1 = plsc.load_gather(vbuf.at[jj], [rows1, icols])
        r = jnp.sum(ua0 * va0 + ua1 * va1)
        acc0 = jnp.where(lane == b, r, acc0)
      return acc0

    fire(0, 0, ua, va, sema)
    lane = lax.broadcasted_iota(jnp.int32, (L,), 0)

    @pl.loop(0, NG)
    def _(gs):
      ws = gs * L
      acc = jnp.zeros((L,), f32)
      for h in range(4):
        bufs = (ua, va, sema) if h % 2 == 0 else (ub, vb, semb)
        nbufs = (ub, vb, semb) if h % 2 == 0 else (ua, va, sema)
        if h < 3:
          fire(ws, h + 1, nbufs[0], nbufs[1], nbufs[2])
        else:
          @pl.when(gs + 1 < NG)
          def _():
            fire(ws + L, 0, nbufs[0], nbufs[1], nbufs[2])
        drain(bufs[0], bufs[1], bufs[2])
        acc = extract(ws, h, bufs[0], bufs[1], acc, lane)
      dotv[pl.ds(ws, L)] = acc

    pltpu.sync_copy(dotv, dot_out.at[pl.ds(base, BPW)])

  return k(user_flat, item_flat, ue, ie)[0]


def _sc_finish(user2d, item2d, dot, user_bias, item_bias, offset_b):
  mesh = plsc.VectorSubcoreMesh(core_axis_name="c", subcore_axis_name="s")
  f32 = jnp.float32

  @functools.partial(
      pl.kernel,
      out_type=jax.ShapeDtypeStruct((B,), f32),
      mesh=mesh,
      compiler_params=pltpu.CompilerParams(use_tc_tiling_on_sc=False),
      scratch_types=[
          pltpu.VMEM((NCH, CH), jnp.int32),
          pltpu.VMEM((NCH, CH), jnp.int32),
          pltpu.VMEM((BPW,), f32),
          pltpu.VMEM((BPW,), f32),
          pltpu.VMEM((BPW,), f32),
          pltpu.VMEM((BPW,), f32),
          pltpu.VMEM((L,), f32),
          pltpu.SemaphoreType.DMA,
      ],
  )
  def k(user_hbm, item_hbm, dot_hbm, ubias_hbm, ibias_hbm, off_hbm, out_hbm,
        uidx_v, iidx_v, ub_v, ib_v, dv_v, outv, off_v, sem):
    cid = lax.axis_index("c")
    sid = lax.axis_index("s")
    wid = sid * NC + cid
    base = wid * BPW
    rowbase = wid * NCH
    pltpu.sync_copy(user_hbm.at[pl.ds(rowbase, NCH)], uidx_v)
    pltpu.sync_copy(item_hbm.at[pl.ds(rowbase, NCH)], iidx_v)
    copies = [pltpu.async_copy(dot_hbm.at[pl.ds(base, BPW)], dv_v, sem)]
    for j in range(NCH):
      copies.append(pltpu.async_copy(
          ubias_hbm.at[uidx_v.at[j]], ub_v.at[pl.ds(j * CH, CH)], sem))
      copies.append(pltpu.async_copy(
          ibias_hbm.at[iidx_v.at[j]], ib_v.at[pl.ds(j * CH, CH)], sem))
    pltpu.sync_copy(off_hbm, off_v)
    for c in copies:
      c.wait()

    @pl.loop(0, BPW, step=L)
    def _(s):
      outv[pl.ds(s, L)] = (dv_v[pl.ds(s, L)] + ub_v[pl.ds(s, L)]
                           + ib_v[pl.ds(s, L)] + off_v[pl.ds(0, L)])

    pltpu.sync_copy(outv, out_hbm.at[pl.ds(base, BPW)])

  return k(user2d, item2d, dot, user_bias, item_bias, offset_b)


@jax.jit
def kernel(user, item, user_emb, item_emb, user_bias, item_bias, offset):
  user = user.astype(jnp.int32)
  item = item.astype(jnp.int32)
  dot = _sc_dot(user, item, user_emb.T, item_emb.T)
  offset_b = jnp.broadcast_to(offset, (L,))
  return _sc_finish(
      user.reshape(IDX_ROWS, CH), item.reshape(IDX_ROWS, CH),
      dot, user_bias, item_bias, offset_b)
